# Initial kernel scaffold; baseline (speedup 1.0000x reference)
import jax
import jax.numpy as jnp
from jax.experimental import pallas as pl


def kernel(x, embedding_weight):
    raise NotImplementedError("write your pallas kernel here")


# speedup vs baseline: 2.8463x; 2.8463x over previous
import jax
import jax.numpy as jnp
from jax.experimental import pallas as pl

SEQ_BLOCK = 512


def _add_kernel(x_ref, w_ref, o_ref):
    o_ref[...] = x_ref[...] + w_ref[...][None, :, :]


def kernel(x, embedding_weight):
    batch, seq_len, hidden = x.shape
    num_blocks = seq_len // SEQ_BLOCK

    return pl.pallas_call(
        _add_kernel,
        grid=(num_blocks, batch),
        in_specs=[
            pl.BlockSpec((1, SEQ_BLOCK, hidden), lambda i, b: (b, i, 0)),
            pl.BlockSpec((SEQ_BLOCK, hidden), lambda i, b: (i, 0)),
        ],
        out_specs=pl.BlockSpec((1, SEQ_BLOCK, hidden), lambda i, b: (b, i, 0)),
        out_shape=jax.ShapeDtypeStruct(x.shape, x.dtype),
    )(x, embedding_weight)


# speedup vs baseline: 3.1739x; 1.1151x over previous
import jax
import jax.numpy as jnp
from jax.experimental import pallas as pl

SEQ_BLOCK = 1024


def _add_kernel(x_ref, w_ref, o_ref):
    o_ref[...] = x_ref[...] + w_ref[...][None, :, :]


def kernel(x, embedding_weight):
    batch, seq_len, hidden = x.shape
    num_blocks = seq_len // SEQ_BLOCK

    return pl.pallas_call(
        _add_kernel,
        grid=(num_blocks, batch),
        in_specs=[
            pl.BlockSpec((1, SEQ_BLOCK, hidden), lambda i, b: (b, i, 0)),
            pl.BlockSpec((SEQ_BLOCK, hidden), lambda i, b: (i, 0)),
        ],
        out_specs=pl.BlockSpec((1, SEQ_BLOCK, hidden), lambda i, b: (b, i, 0)),
        out_shape=jax.ShapeDtypeStruct(x.shape, x.dtype),
    )(x, embedding_weight)


# speedup vs baseline: 3.3096x; 1.0428x over previous
import jax
import jax.numpy as jnp
from jax.experimental import pallas as pl

SEQ_BLOCK = 2048


def _add_kernel(x_ref, w_ref, o_ref):
    o_ref[...] = x_ref[...] + w_ref[...][None, :, :]


def kernel(x, embedding_weight):
    batch, seq_len, hidden = x.shape
    num_blocks = seq_len // SEQ_BLOCK

    return pl.pallas_call(
        _add_kernel,
        grid=(num_blocks, batch),
        in_specs=[
            pl.BlockSpec((1, SEQ_BLOCK, hidden), lambda i, b: (b, i, 0)),
            pl.BlockSpec((SEQ_BLOCK, hidden), lambda i, b: (i, 0)),
        ],
        out_specs=pl.BlockSpec((1, SEQ_BLOCK, hidden), lambda i, b: (b, i, 0)),
        out_shape=jax.ShapeDtypeStruct(x.shape, x.dtype),
    )(x, embedding_weight)


# speedup vs baseline: 3.3108x; 1.0004x over previous
import jax
import jax.numpy as jnp
from jax.experimental import pallas as pl
from jax.experimental.pallas import tpu as pltpu

SEQ_BLOCK = 2048


def _add_kernel(x_ref, w_ref, o_ref):
    o_ref[...] = x_ref[...] + w_ref[...][None, :, :]


def kernel(x, embedding_weight):
    batch, seq_len, hidden = x.shape
    num_blocks = seq_len // SEQ_BLOCK

    return pl.pallas_call(
        _add_kernel,
        grid=(num_blocks, batch),
        in_specs=[
            pl.BlockSpec((1, SEQ_BLOCK, hidden), lambda i, b: (b, i, 0)),
            pl.BlockSpec((SEQ_BLOCK, hidden), lambda i, b: (i, 0)),
        ],
        out_specs=pl.BlockSpec((1, SEQ_BLOCK, hidden), lambda i, b: (b, i, 0)),
        out_shape=jax.ShapeDtypeStruct(x.shape, x.dtype),
        compiler_params=pltpu.CompilerParams(
            dimension_semantics=("parallel", "parallel"),
        ),
    )(x, embedding_weight)
